# Initial kernel scaffold; baseline (speedup 1.0000x reference)
import jax
import jax.numpy as jnp
from jax.experimental import pallas as pl


def kernel(x, edge_index, batch, emb0, emb1, emb2, emb3, emb4, emb5, emb6, emb7, emb8, W1, b1, W2, b2, eps, gamma, beta, pW1, pb1, pW2, pb2):
    raise NotImplementedError("write your pallas kernel here")


# speedup vs baseline: 2.6201x; 2.6201x over previous
import functools

import jax
import jax.numpy as jnp
from jax import lax
from jax.experimental import pallas as pl
from jax.experimental.pallas import tpu as pltpu
from jax.experimental.pallas import tpu_sc as plsc

N = 10000
H = 128
E = 320000
NG = 256
NLAYERS = 5
PROJ = 256

NC = 2
NS = 16
NW = NC * NS
NPAD = 10240
EW = 10240
EPAD = NW * EW
CHUNK = 128
NCHUNK = EW // CHUNK
ROWS_PER_TILE = NPAD // NS

ATOM_DIMS = (119, 4, 12, 12, 10, 6, 6, 2, 2)
VOCAB = 256

def _sc_aggregate_body(h_hbm, src_hbm, dst_hbm, out_hbm,
                       src_v, dst_v, rows_v, zero_v, acc_sh, sem):
    cid = lax.axis_index("c")
    sid = lax.axis_index("s")
    wid = cid * NS + sid

    zero16 = jnp.zeros((16,), jnp.float32)

    def _zfill(r, c):
        for j in range(H // 16):
            zero_v[r, pl.ds(j * 16, 16)] = zero16
        return c

    lax.fori_loop(0, 64, _zfill, 0)

    def _zcopy(i, c):
        pltpu.sync_copy(zero_v, acc_sh.at[pl.ds(sid * ROWS_PER_TILE + i * 64, 64)])
        return c

    lax.fori_loop(0, ROWS_PER_TILE // 64, _zcopy, 0)
    plsc.subcore_barrier()

    ebase = wid * EW

    def _edge_chunk(ci, c):
        b = ebase + ci * CHUNK
        pltpu.sync_copy(src_hbm.at[pl.ds(b, CHUNK)], src_v)
        pltpu.sync_copy(dst_hbm.at[pl.ds(b, CHUNK)], dst_v)
        pltpu.async_copy(h_hbm.at[src_v], rows_v, sem).wait()
        pltpu.sync_copy(rows_v, acc_sh.at[dst_v], add=True)
        return c

    lax.fori_loop(0, NCHUNK, _edge_chunk, 0)
    plsc.subcore_barrier()

    r0 = sid * ROWS_PER_TILE
    pltpu.sync_copy(acc_sh.at[pl.ds(r0, ROWS_PER_TILE)],
                    out_hbm.at[cid, pl.ds(r0, ROWS_PER_TILE)])


@functools.lru_cache(maxsize=None)
def _sc_aggregate_kernel():
    mesh = plsc.VectorSubcoreMesh(core_axis_name="c", subcore_axis_name="s",
                                  num_cores=NC, num_subcores=NS)
    return pl.kernel(
        _sc_aggregate_body,
        out_type=jax.ShapeDtypeStruct((NC, NPAD, H), jnp.float32),
        mesh=mesh,
        scratch_types=[
            pltpu.VMEM((CHUNK,), jnp.int32),
            pltpu.VMEM((CHUNK,), jnp.int32),
            pltpu.VMEM((CHUNK, H), jnp.float32),
            pltpu.VMEM((64, H), jnp.float32),
            pltpu.VMEM_SHARED((NPAD, H), jnp.float32),
            pltpu.SemaphoreType.DMA,
        ],
    )


def _tc_encode_body(xo_ref, tab_ref, out_ref):
    xo = xo_ref[...]
    iota = lax.broadcasted_iota(jnp.int32, (1, VOCAB), 1)
    m = jnp.zeros((N, VOCAB), jnp.float32)
    for i in range(9):
        m = m + (xo[:, i:i + 1] == iota).astype(jnp.float32)
    out_ref[...] = jnp.dot(m, tab_ref[...], preferred_element_type=jnp.float32, precision=lax.Precision.HIGHEST)


def _tc_layer_body(h_ref, acc_ref, w1_ref, b1_ref, w2_ref, b2_ref,
                   eps_ref, gamma_ref, beta_ref, out_ref):
    h = h_ref[...]
    z = h * (1.0 + eps_ref[0, 0]) + acc_ref[0, :N, :] + acc_ref[1, :N, :]
    def _rbf(a):
        return a.astype(jnp.bfloat16).astype(jnp.float32)
    t = jnp.maximum(jnp.dot(_rbf(z), _rbf(w1_ref[...]),
                            preferred_element_type=jnp.float32)
                    + b1_ref[...], 0.0)
    z2 = jnp.dot(_rbf(t), _rbf(w2_ref[...]),
                 preferred_element_type=jnp.float32) + b2_ref[...]
    mean = jnp.mean(z2, axis=0, keepdims=True)
    var = jnp.mean((z2 - mean) ** 2, axis=0, keepdims=True)
    zn = (z2 - mean) / jnp.sqrt(var + 1e-5) * gamma_ref[...] + beta_ref[...]
    out_ref[...] = jnp.maximum(zn, 0.0)


def _tc_pool_body(h_ref, b_ref, pw1_ref, pb1_ref, pw2_ref, pb2_ref, out_ref):
    bcol = b_ref[...]
    iota = lax.broadcasted_iota(jnp.int32, (1, NG), 1)
    oh = (bcol == iota).astype(jnp.float32)
    sums = lax.dot_general(oh, h_ref[...], (((0,), (0,)), ((), ())),
                           preferred_element_type=jnp.float32, precision=lax.Precision.HIGHEST)
    ones = jnp.ones((N, 1), jnp.float32)
    counts = lax.dot_general(oh, ones, (((0,), (0,)), ((), ())),
                             preferred_element_type=jnp.float32, precision=lax.Precision.HIGHEST)
    g = sums / jnp.maximum(counts, 1.0)
    p1 = jnp.maximum(jnp.dot(g, pw1_ref[...], preferred_element_type=jnp.float32, precision=lax.Precision.HIGHEST)
                     + pb1_ref[...], 0.0)
    p = jnp.dot(p1, pw2_ref[...], preferred_element_type=jnp.float32, precision=lax.Precision.HIGHEST) + pb2_ref[...]
    nrm = jnp.sqrt(jnp.sum(p * p, axis=1, keepdims=True))
    out_ref[...] = p / jnp.maximum(nrm, 1e-12)


_tc_encode = pl.pallas_call(
    _tc_encode_body,
    out_shape=jax.ShapeDtypeStruct((N, H), jnp.float32),
)

_tc_layer = pl.pallas_call(
    _tc_layer_body,
    out_shape=jax.ShapeDtypeStruct((N, H), jnp.float32),
)

_tc_pool = pl.pallas_call(
    _tc_pool_body,
    out_shape=jax.ShapeDtypeStruct((NG, H), jnp.float32),
)


def kernel(x, edge_index, batch, emb0, emb1, emb2, emb3, emb4, emb5, emb6,
           emb7, emb8, W1, b1, W2, b2, eps, gamma, beta, pW1, pb1, pW2, pb2):
    embs = (emb0, emb1, emb2, emb3, emb4, emb5, emb6, emb7, emb8)

    offs = []
    o = 0
    for d in ATOM_DIMS:
        offs.append(o)
        o += d
    xo = x.astype(jnp.int32) + jnp.asarray(offs, jnp.int32)[None, :]
    xo = jnp.concatenate(
        [xo, jnp.full((N, 16 - 9), -1, jnp.int32)], axis=1)
    tab = jnp.concatenate(list(embs), axis=0)
    tab = jnp.concatenate(
        [tab, jnp.zeros((VOCAB - tab.shape[0], H), jnp.float32)], axis=0)

    src = edge_index[0].astype(jnp.int32)
    dst = edge_index[1].astype(jnp.int32)
    pad = EPAD - E
    src = jnp.concatenate([src, jnp.zeros((pad,), jnp.int32)])
    dst = jnp.concatenate([dst, jnp.full((pad,), N, jnp.int32)])

    bcol = batch.astype(jnp.int32)[:, None]

    h = _tc_encode(xo, tab)
    for l in range(NLAYERS):
        acc = _sc_aggregate_kernel()(h, src, dst)
        h = _tc_layer(h, acc, W1[l], b1[l][None, :], W2[l], b2[l][None, :],
                      eps[l][None, None], gamma[l][None, :], beta[l][None, :])
    return _tc_pool(h, bcol, pW1, pb1[None, :], pW2, pb2[None, :])


# speedup vs baseline: 3.2951x; 1.2576x over previous
import functools

import jax
import jax.numpy as jnp
from jax import lax
from jax.experimental import pallas as pl
from jax.experimental.pallas import tpu as pltpu
from jax.experimental.pallas import tpu_sc as plsc

N = 10000
H = 128
E = 320000
NG = 256
NLAYERS = 5
PROJ = 256

NC = 2
NS = 16
NW = NC * NS
NPAD = 10112
ROWS_PER_TILE = NPAD // NS
EW = 10240
EPAD = NW * EW
CHUNK = 128
NCHUNK = EW // CHUNK
NPHASE = 2
PHASE = NCHUNK // NPHASE
NBUF = 2

ATOM_DIMS = (119, 4, 12, 12, 10, 6, 6, 2, 2)
VOCAB = 256


def _sc_aggregate_body(h_hbm, src_hbm, dst_hbm, out_hbm,
                       srcs_v, dsts_v, rows_v, acc_sh, sem0, sem1):
    cid = lax.axis_index("c")
    sid = lax.axis_index("s")
    wid = cid * NS + sid
    sems = (sem0, sem1)

    zero16 = jnp.zeros((16,), jnp.float32)

    def _zfill(r, c):
        for j in range(H // 16):
            rows_v[0, r, pl.ds(j * 16, 16)] = zero16
        return c

    lax.fori_loop(0, CHUNK, _zfill, 0)
    base_r = sid * ROWS_PER_TILE
    for i in range(ROWS_PER_TILE // CHUNK):
        pltpu.sync_copy(rows_v.at[0],
                        acc_sh.at[pl.ds(base_r + i * CHUNK, CHUNK)])
    rem = ROWS_PER_TILE % CHUNK
    pltpu.sync_copy(rows_v.at[0, pl.ds(0, rem)],
                    acc_sh.at[pl.ds(base_r + ROWS_PER_TILE - rem, rem)])
    plsc.subcore_barrier()

    for ph in range(NPHASE):
        pltpu.sync_copy(src_hbm.at[wid, ph], srcs_v)
        pltpu.sync_copy(dst_hbm.at[wid, ph], dsts_v)
        for b in range(NBUF):
            pltpu.async_copy(h_hbm.at[srcs_v.at[b]], rows_v.at[b], sems[b])

        ngrp = PHASE // NBUF

        def _group(gi, c):
            for b in range(NBUF):
                j = gi * NBUF + b
                pltpu.make_async_copy(h_hbm.at[srcs_v.at[j]], rows_v.at[b],
                                      sems[b]).wait()
                pltpu.sync_copy(rows_v.at[b], acc_sh.at[dsts_v.at[j]],
                                add=True)

                @pl.when(gi + 1 < ngrp)
                def _():
                    pltpu.async_copy(h_hbm.at[srcs_v.at[j + NBUF]],
                                     rows_v.at[b], sems[b])
            return c

        lax.fori_loop(0, ngrp, _group, 0)

    plsc.subcore_barrier()
    pltpu.sync_copy(acc_sh.at[pl.ds(base_r, ROWS_PER_TILE)],
                    out_hbm.at[cid, pl.ds(base_r, ROWS_PER_TILE)])


@functools.lru_cache(maxsize=None)
def _sc_aggregate_kernel():
    mesh = plsc.VectorSubcoreMesh(core_axis_name="c", subcore_axis_name="s",
                                  num_cores=NC, num_subcores=NS)
    return pl.kernel(
        _sc_aggregate_body,
        out_type=jax.ShapeDtypeStruct((NC, NPAD, H), jnp.float32),
        mesh=mesh,
        scratch_types=[
            pltpu.VMEM((PHASE, CHUNK), jnp.int32),
            pltpu.VMEM((PHASE, CHUNK), jnp.int32),
            pltpu.VMEM((NBUF, CHUNK, H), jnp.float32),
            pltpu.VMEM_SHARED((NPAD, H), jnp.float32),
            pltpu.SemaphoreType.DMA,
            pltpu.SemaphoreType.DMA,
        ],
    )


def _tc_encode_body(xo_ref, tab_ref, out_ref):
    xo = xo_ref[...]
    iota = lax.broadcasted_iota(jnp.int32, (1, VOCAB), 1)
    m = jnp.zeros((N, VOCAB), jnp.float32)
    for i in range(9):
        m = m + (xo[:, i:i + 1] == iota).astype(jnp.float32)
    out_ref[...] = jnp.dot(m, tab_ref[...], preferred_element_type=jnp.float32,
                           precision=lax.Precision.HIGHEST)


def _tc_layer_body(h_ref, acc_ref, w1_ref, b1_ref, w2_ref, b2_ref,
                   eps_ref, gamma_ref, beta_ref, out_ref):
    h = h_ref[...]
    z = h * (1.0 + eps_ref[0, 0]) + acc_ref[0, :N, :] + acc_ref[1, :N, :]

    def _rbf(a):
        return a.astype(jnp.bfloat16).astype(jnp.float32)

    t = jnp.maximum(jnp.dot(_rbf(z), _rbf(w1_ref[...]),
                            preferred_element_type=jnp.float32)
                    + b1_ref[...], 0.0)
    z2 = jnp.dot(_rbf(t), _rbf(w2_ref[...]),
                 preferred_element_type=jnp.float32) + b2_ref[...]
    mean = jnp.mean(z2, axis=0, keepdims=True)
    var = jnp.mean((z2 - mean) ** 2, axis=0, keepdims=True)
    zn = (z2 - mean) / jnp.sqrt(var + 1e-5) * gamma_ref[...] + beta_ref[...]
    out_ref[...] = jnp.maximum(zn, 0.0)


def _tc_pool_body(h_ref, b_ref, pw1_ref, pb1_ref, pw2_ref, pb2_ref, out_ref):
    bcol = b_ref[...]
    iota = lax.broadcasted_iota(jnp.int32, (1, NG), 1)
    oh = (bcol == iota).astype(jnp.float32)
    sums = lax.dot_general(oh, h_ref[...], (((0,), (0,)), ((), ())),
                           preferred_element_type=jnp.float32,
                           precision=lax.Precision.HIGHEST)
    ones = jnp.ones((N, 1), jnp.float32)
    counts = lax.dot_general(oh, ones, (((0,), (0,)), ((), ())),
                             preferred_element_type=jnp.float32,
                             precision=lax.Precision.HIGHEST)
    g = sums / jnp.maximum(counts, 1.0)
    p1 = jnp.maximum(jnp.dot(g, pw1_ref[...],
                             preferred_element_type=jnp.float32,
                             precision=lax.Precision.HIGHEST)
                     + pb1_ref[...], 0.0)
    p = jnp.dot(p1, pw2_ref[...], preferred_element_type=jnp.float32,
                precision=lax.Precision.HIGHEST) + pb2_ref[...]
    nrm = jnp.sqrt(jnp.sum(p * p, axis=1, keepdims=True))
    out_ref[...] = p / jnp.maximum(nrm, 1e-12)


_tc_encode = pl.pallas_call(
    _tc_encode_body,
    out_shape=jax.ShapeDtypeStruct((N, H), jnp.float32),
)

_tc_layer = pl.pallas_call(
    _tc_layer_body,
    out_shape=jax.ShapeDtypeStruct((N, H), jnp.float32),
)

_tc_pool = pl.pallas_call(
    _tc_pool_body,
    out_shape=jax.ShapeDtypeStruct((NG, H), jnp.float32),
)


def kernel(x, edge_index, batch, emb0, emb1, emb2, emb3, emb4, emb5, emb6,
           emb7, emb8, W1, b1, W2, b2, eps, gamma, beta, pW1, pb1, pW2, pb2):
    embs = (emb0, emb1, emb2, emb3, emb4, emb5, emb6, emb7, emb8)

    offs = []
    o = 0
    for d in ATOM_DIMS:
        offs.append(o)
        o += d
    xo = x.astype(jnp.int32) + jnp.asarray(offs, jnp.int32)[None, :]
    xo = jnp.concatenate(
        [xo, jnp.full((N, 16 - 9), -1, jnp.int32)], axis=1)
    tab = jnp.concatenate(list(embs), axis=0)
    tab = jnp.concatenate(
        [tab, jnp.zeros((VOCAB - tab.shape[0], H), jnp.float32)], axis=0)

    src = edge_index[0].astype(jnp.int32)
    dst = edge_index[1].astype(jnp.int32)
    pad = EPAD - E
    src = jnp.concatenate([src, jnp.zeros((pad,), jnp.int32)])
    dst = jnp.concatenate([dst, jnp.full((pad,), N, jnp.int32)])
    src = src.reshape(NW, NPHASE, PHASE, CHUNK)
    dst = dst.reshape(NW, NPHASE, PHASE, CHUNK)

    bcol = batch.astype(jnp.int32)[:, None]

    h = _tc_encode(xo, tab)
    for l in range(NLAYERS):
        acc = _sc_aggregate_kernel()(h, src, dst)
        h = _tc_layer(h, acc, W1[l], b1[l][None, :], W2[l], b2[l][None, :],
                      eps[l][None, None], gamma[l][None, :], beta[l][None, :])
    return _tc_pool(h, bcol, pW1, pb1[None, :], pW2, pb2[None, :])
